# baseline (speedup 1.0000x reference)
import functools

import jax
import jax.numpy as jnp
from jax.experimental import pallas as pl
from jax.experimental.pallas import tpu as pltpu

_BN = 512


def _chamfer_body(a_ref, b_ref, out_ref, colmax_ref, acc_ref):
    i = pl.program_id(0)
    nsteps = pl.num_programs(0)

    ablk = a_ref[...]
    bmat = b_ref[...]

    g = jnp.dot(ablk, bmat, preferred_element_type=jnp.float32)

    rowmax_sum = jnp.sum(jnp.max(g, axis=1))
    colmax = jnp.max(g, axis=0, keepdims=True)

    @pl.when(i == 0)
    def _init():
        acc_ref[0, 0] = rowmax_sum
        colmax_ref[...] = colmax

    @pl.when(i > 0)
    def _update():
        acc_ref[0, 0] += rowmax_sum
        colmax_ref[...] = jnp.maximum(colmax_ref[...], colmax)

    @pl.when(i == nsteps - 1)
    def _finish():
        n = nsteps * ablk.shape[0]
        m = bmat.shape[1]
        loss = -2.0 * (acc_ref[0, 0] / n + jnp.sum(colmax_ref[...]) / m)
        out_ref[...] = jnp.reshape(loss, (1, 1))


def _split_hi_lo(x):
    hi = jax.lax.optimization_barrier(x.astype(jnp.bfloat16))
    lo = jax.lax.optimization_barrier(
        (x - hi.astype(jnp.float32)).astype(jnp.bfloat16))
    return hi, lo


@functools.partial(jax.jit, static_argnames=())
def kernel(predict, target):
    p = predict[0]
    t = target[0]
    n, _ = p.shape
    m, _ = t.shape

    p_hi, p_lo = _split_hi_lo(p)
    t_hi, t_lo = _split_hi_lo(t)
    p_rec = p_hi.astype(jnp.float32) + p_lo.astype(jnp.float32)
    t_rec = t_hi.astype(jnp.float32) + t_lo.astype(jnp.float32)
    pn = 0.5 * jnp.sum(p_rec * p_rec, axis=1, keepdims=True)
    tn = 0.5 * jnp.sum(t_rec * t_rec, axis=1, keepdims=True)
    pn_hi, pn_lo = _split_hi_lo(pn)
    tn_hi, tn_lo = _split_hi_lo(tn)

    one_n = jnp.ones((n, 1), jnp.bfloat16)
    one_m = jnp.ones((m, 1), jnp.bfloat16)
    a = jnp.concatenate(
        [p_hi, p_hi, p_lo, p_lo, pn_hi, pn_lo, one_n, one_n], axis=1)
    b = jnp.concatenate(
        [t_hi, t_lo, t_hi, t_lo, -one_m, -one_m, -tn_hi, -tn_lo], axis=1).T

    out = pl.pallas_call(
        _chamfer_body,
        grid=(n // _BN,),
        in_specs=[
            pl.BlockSpec((_BN, 16), lambda i: (i, 0)),
            pl.BlockSpec((16, m), lambda i: (0, 0)),
        ],
        out_specs=pl.BlockSpec((1, 1), lambda i: (0, 0)),
        out_shape=jax.ShapeDtypeStruct((1, 1), jnp.float32),
        scratch_shapes=[
            pltpu.VMEM((1, m), jnp.float32),
            pltpu.SMEM((1, 1), jnp.float32),
        ],
    )(a, b)
    return out[0, 0]


# speedup vs baseline: 1.0607x; 1.0607x over previous
import functools

import jax
import jax.numpy as jnp
from jax.experimental import pallas as pl
from jax.experimental.pallas import tpu as pltpu

_BN = 1024


def _chamfer_body(a_ref, b_ref, out_ref, colmax_ref, acc_ref):
    i = pl.program_id(0)
    nsteps = pl.num_programs(0)

    ablk = a_ref[...]
    m = b_ref.shape[1]

    g = jnp.dot(ablk, b_ref[...],
                preferred_element_type=jnp.float32)

    rowmax = jnp.max(g, axis=1, keepdims=True)
    rowmax_sum = jnp.sum(rowmax)
    colmax = jnp.max(g, axis=0, keepdims=True)

    @pl.when(i == 0)
    def _init():
        acc_ref[0, 0] = rowmax_sum
        colmax_ref[...] = colmax

    @pl.when(i > 0)
    def _update():
        acc_ref[0, 0] += rowmax_sum
        colmax_ref[...] = jnp.maximum(colmax_ref[...], colmax)

    @pl.when(i == nsteps - 1)
    def _finish():
        n = nsteps * ablk.shape[0]
        csum = jnp.sum(colmax_ref[...])
        loss = -2.0 * (acc_ref[0, 0] / n + csum / m)
        out_ref[...] = jnp.reshape(loss, (1, 1))


def _split_hi_lo(x):
    hi = jax.lax.optimization_barrier(x.astype(jnp.bfloat16))
    lo = jax.lax.optimization_barrier(
        (x - hi.astype(jnp.float32)).astype(jnp.bfloat16))
    return hi, lo


@functools.partial(jax.jit, static_argnames=())
def kernel(predict, target):
    p = predict[0]
    t = target[0]
    n, _ = p.shape
    m, _ = t.shape

    p_hi, p_lo = _split_hi_lo(p)
    t_hi, t_lo = _split_hi_lo(t)
    p_rec = p_hi.astype(jnp.float32) + p_lo.astype(jnp.float32)
    t_rec = t_hi.astype(jnp.float32) + t_lo.astype(jnp.float32)
    pn = 0.5 * jnp.sum(p_rec * p_rec, axis=1, keepdims=True)
    tn = 0.5 * jnp.sum(t_rec * t_rec, axis=1, keepdims=True)
    pn_hi, pn_lo = _split_hi_lo(pn)
    tn_hi, tn_lo = _split_hi_lo(tn)

    one_n = jnp.ones((n, 1), jnp.bfloat16)
    one_m = jnp.ones((m, 1), jnp.bfloat16)
    a = jnp.concatenate(
        [p_hi, p_hi, p_lo, p_lo, pn_hi, pn_lo, one_n, one_n], axis=1)
    b = jnp.concatenate(
        [t_hi, t_lo, t_hi, t_lo, -one_m, -one_m, -tn_hi, -tn_lo], axis=1).T

    out = pl.pallas_call(
        _chamfer_body,
        grid=(n // _BN,),
        in_specs=[
            pl.BlockSpec((_BN, 16), lambda i: (i, 0)),
            pl.BlockSpec((16, m), lambda i: (0, 0)),
        ],
        out_specs=pl.BlockSpec((1, 1), lambda i: (0, 0)),
        out_shape=jax.ShapeDtypeStruct((1, 1), jnp.float32),
        scratch_shapes=[
            pltpu.VMEM((1, m), jnp.float32),
            pltpu.SMEM((1, 1), jnp.float32),
        ],
    )(a, b)
    return out[0, 0]


# speedup vs baseline: 1.0902x; 1.0277x over previous
import functools

import jax
import jax.numpy as jnp
from jax.experimental import pallas as pl
from jax.experimental.pallas import tpu as pltpu

_BN = 1024


def _chamfer_body(a_ref, b_ref, out_ref, colmax_ref, acc_ref):
    i = pl.program_id(0)
    nsteps = pl.num_programs(0)

    ablk = a_ref[...]
    m = b_ref.shape[1]

    g = jnp.dot(ablk, b_ref[...],
                preferred_element_type=jnp.float32)

    rowmax = jnp.max(g, axis=1, keepdims=True)
    rowmax_sum = jnp.sum(rowmax)
    colmax = jnp.max(g, axis=0, keepdims=True)

    @pl.when(i == 0)
    def _init():
        acc_ref[0, 0] = rowmax_sum
        colmax_ref[...] = colmax

    @pl.when(i > 0)
    def _update():
        acc_ref[0, 0] += rowmax_sum
        colmax_ref[...] = jnp.maximum(colmax_ref[...], colmax)

    @pl.when(i == nsteps - 1)
    def _finish():
        n = nsteps * ablk.shape[0]
        csum = jnp.sum(colmax_ref[...])
        loss = -2.0 * (acc_ref[0, 0] / n + csum / m)
        out_ref[...] = jnp.reshape(loss, (1, 1))


def _split_hi_lo(x):
    bits = jax.lax.bitcast_convert_type(x, jnp.uint32)
    hi32 = jax.lax.bitcast_convert_type(
        bits & jnp.uint32(0xFFFF0000), jnp.float32)
    hi = hi32.astype(jnp.bfloat16)
    lo = (x - hi32).astype(jnp.bfloat16)
    return hi, lo


@functools.partial(jax.jit, static_argnames=())
def kernel(predict, target):
    p = predict[0]
    t = target[0]
    n, _ = p.shape
    m, _ = t.shape

    p_hi, p_lo = _split_hi_lo(p)
    t_hi, t_lo = _split_hi_lo(t)
    pn = 0.5 * jnp.sum(p * p, axis=1, keepdims=True)
    tn = 0.5 * jnp.sum(t * t, axis=1, keepdims=True)
    pn_hi, pn_lo = _split_hi_lo(pn)
    tn_hi, tn_lo = _split_hi_lo(tn)

    one_n = jnp.ones((n, 1), jnp.bfloat16)
    one_m = jnp.ones((m, 1), jnp.bfloat16)
    a = jnp.concatenate(
        [p_hi, p_hi, p_lo, p_lo, pn_hi, pn_lo, one_n, one_n], axis=1)
    b = jnp.concatenate(
        [t_hi, t_lo, t_hi, t_lo, -one_m, -one_m, -tn_hi, -tn_lo], axis=1).T

    out = pl.pallas_call(
        _chamfer_body,
        grid=(n // _BN,),
        in_specs=[
            pl.BlockSpec((_BN, 16), lambda i: (i, 0)),
            pl.BlockSpec((16, m), lambda i: (0, 0)),
        ],
        out_specs=pl.BlockSpec((1, 1), lambda i: (0, 0)),
        out_shape=jax.ShapeDtypeStruct((1, 1), jnp.float32),
        scratch_shapes=[
            pltpu.VMEM((1, m), jnp.float32),
            pltpu.SMEM((1, 1), jnp.float32),
        ],
    )(a, b)
    return out[0, 0]


# speedup vs baseline: 1.0932x; 1.0028x over previous
import functools

import jax
import jax.numpy as jnp
from jax.experimental import pallas as pl
from jax.experimental.pallas import tpu as pltpu

_BN = 1024
_NCHUNKS = 4


def _chamfer_body(a_ref, b_ref, out_ref, colmax_ref, acc_ref):
    i = pl.program_id(0)
    nsteps = pl.num_programs(0)
    m = b_ref.shape[1]
    ch = m // _NCHUNKS

    @pl.when(i == 0)
    def _init():
        acc_ref[0, 0] = 0.0
        colmax_ref[...] = jnp.full((1, m), -jnp.inf, jnp.float32)

    ablk = a_ref[...]

    rowmax = None
    for c in range(_NCHUNKS):
        sl = pl.ds(c * ch, ch)
        g = jnp.dot(ablk, b_ref[:, sl],
                    preferred_element_type=jnp.float32)
        rm = jnp.max(g, axis=1, keepdims=True)
        rowmax = rm if rowmax is None else jnp.maximum(rowmax, rm)
        colmax_ref[:, sl] = jnp.maximum(colmax_ref[:, sl],
                                        jnp.max(g, axis=0, keepdims=True))
    acc_ref[0, 0] += jnp.sum(rowmax)

    @pl.when(i == nsteps - 1)
    def _finish():
        n = nsteps * ablk.shape[0]
        csum = jnp.sum(colmax_ref[...])
        loss = -2.0 * (acc_ref[0, 0] / n + csum / m)
        out_ref[...] = jnp.reshape(loss, (1, 1))


def _split_hi_lo(x):
    bits = jax.lax.bitcast_convert_type(x, jnp.uint32)
    hi32 = jax.lax.bitcast_convert_type(
        bits & jnp.uint32(0xFFFF0000), jnp.float32)
    hi = hi32.astype(jnp.bfloat16)
    lo = (x - hi32).astype(jnp.bfloat16)
    return hi, lo


@functools.partial(jax.jit, static_argnames=())
def kernel(predict, target):
    p = predict[0]
    t = target[0]
    n, _ = p.shape
    m, _ = t.shape

    p_hi, p_lo = _split_hi_lo(p)
    t_hi, t_lo = _split_hi_lo(t)
    pn = 0.5 * jnp.sum(p * p, axis=1, keepdims=True)
    tn = 0.5 * jnp.sum(t * t, axis=1, keepdims=True)
    pn_hi, pn_lo = _split_hi_lo(pn)
    tn_hi, tn_lo = _split_hi_lo(tn)

    one_n = jnp.ones((n, 1), jnp.bfloat16)
    one_m = jnp.ones((m, 1), jnp.bfloat16)
    a = jnp.concatenate(
        [p_hi, p_hi, p_lo, p_lo, pn_hi, pn_lo, one_n, one_n], axis=1)
    b = jnp.concatenate(
        [t_hi, t_lo, t_hi, t_lo, -one_m, -one_m, -tn_hi, -tn_lo], axis=1).T

    out = pl.pallas_call(
        _chamfer_body,
        grid=(n // _BN,),
        in_specs=[
            pl.BlockSpec((_BN, 16), lambda i: (i, 0)),
            pl.BlockSpec((16, m), lambda i: (0, 0)),
        ],
        out_specs=pl.BlockSpec((1, 1), lambda i: (0, 0)),
        out_shape=jax.ShapeDtypeStruct((1, 1), jnp.float32),
        scratch_shapes=[
            pltpu.VMEM((1, m), jnp.float32),
            pltpu.SMEM((1, 1), jnp.float32),
        ],
    )(a, b)
    return out[0, 0]
